# baseline (speedup 1.0000x reference)
import functools

import jax
import jax.numpy as jnp
from jax import lax
from jax.experimental import pallas as pl
from jax.experimental.pallas import tpu as pltpu
from jax.experimental.pallas import tpu_sc as plsc

N_NODES = 10000
N_EDGES = 320000
N_GRAPHS = 16
D_FEAT = 128
D_EDGE = 16
LATENT = 16

NC = 2
NS = 16
NW = NC * NS
LANE = 16

EPW = N_EDGES // NW
STEP = 128
G = 3
NGRP = 26
NPAIR = NGRP // 2
GROUP_E = G * STEP
TAIL = EPW - NGRP * GROUP_E
VPG = GROUP_E // LANE


def _node_pre_body(x8_ref, wbr_ref, wbs_ref, u_ref, wu_ref, b8_ref, r_ref,
                   m8_ref, prer_ref, pres_ref):
    x8 = x8_ref[...]
    prer_ref[...] = jnp.dot(x8, wbr_ref[...], preferred_element_type=jnp.float32)
    uw = jnp.dot(u_ref[...], wu_ref[...], preferred_element_type=jnp.float32)
    uw8 = jnp.tile(uw, (8, 8)) * m8_ref[...]
    brep = jnp.dot(b8_ref[...].astype(jnp.float32), r_ref[...],
                   preferred_element_type=jnp.float32)
    g16 = (lax.broadcasted_iota(jnp.int32, (1, 128), 1) % 16).astype(jnp.float32)
    oh = (brep == g16).astype(jnp.float32)
    pres_ref[...] = (jnp.dot(x8, wbs_ref[...], preferred_element_type=jnp.float32)
                     + jnp.dot(oh, uw8, preferred_element_type=jnp.float32))


def _node_pre(x8, wbr, wbs, u, wu, b8, r, m8):
    return pl.pallas_call(
        _node_pre_body,
        out_shape=(
            jax.ShapeDtypeStruct((N_NODES // 8, 128), jnp.float32),
            jax.ShapeDtypeStruct((N_NODES // 8, 128), jnp.float32),
        ),
    )(x8, wbr, wbs, u, wu, b8, r, m8)


def _gather_body(prer_hbm, pres_hbm, col_hbm, row_hbm, ghi_hbm, glo_hbm,
                 colv, rowv, bufr0, bufs0, bufr1, bufs1, bufr2, bufs2,
                 bhi0, blo0, bhi1, blo1, bhi2, blo2,
                 semr0, sems0, semr1, sems1, semr2, sems2,
                 semw0, semw1, semw2):
    wid = lax.axis_index("s") * NC + lax.axis_index("c")
    ebase = wid * EPW
    pltpu.sync_copy(col_hbm.at[pl.ds(ebase, EPW)], colv)
    pltpu.sync_copy(row_hbm.at[pl.ds(ebase, EPW)], rowv)
    lane = lax.broadcasted_iota(jnp.int32, (LANE,), 0)

    def fire(g, bufr, bufs, semr, sems):
        for b in range(G):
            o = g * GROUP_E + b * STEP
            d = pl.ds(b * STEP, STEP)
            pltpu.async_copy(prer_hbm.at[colv.at[pl.ds(o, STEP)]],
                             bufr.at[d], semr)
            pltpu.async_copy(pres_hbm.at[rowv.at[pl.ds(o, STEP)]],
                             bufs.at[d], sems)

    def drain_gathers(bufr, bufs, semr, sems):
        pltpu.make_async_copy(prer_hbm.at[pl.ds(0, GROUP_E)], bufr, semr).wait()
        pltpu.make_async_copy(pres_hbm.at[pl.ds(0, GROUP_E)], bufs, sems).wait()

    def transpose(bufr, bufs, bhi, blo, n_vregs):
        def make_frow(half, fofs):
            def frow(f):
                fcol = jnp.full((LANE,), f + fofs, jnp.int32)
                vs = []
                for b in range(n_vregs):
                    ridx = lane + (LANE * b)
                    vs.append(plsc.load_gather(bufr, [ridx, fcol])
                              + plsc.load_gather(bufs, [ridx, fcol]))
                for b in range(n_vregs):
                    half[f, pl.ds(b * LANE, LANE)] = vs[b]
            return frow
        plsc.parallel_loop(0, 8)(make_frow(bhi, 0))
        plsc.parallel_loop(0, 8)(make_frow(blo, 8))

    def write(bhi, blo, g, semw):
        d = pl.ds(ebase + g * GROUP_E, GROUP_E)
        pltpu.async_copy(bhi, ghi_hbm.at[:, d], semw)
        pltpu.async_copy(blo, glo_hbm.at[:, d], semw)

    def drain_write(bhi, blo, semw):
        d = pl.ds(ebase, GROUP_E)
        pltpu.make_async_copy(bhi, ghi_hbm.at[:, d], semw).wait()
        pltpu.make_async_copy(blo, glo_hbm.at[:, d], semw).wait()

    sets = (
        (bufr0, bufs0, bhi0, blo0, semr0, sems0, semw0),
        (bufr1, bufs1, bhi1, blo1, semr1, sems1, semw1),
        (bufr2, bufs2, bhi2, blo2, semr2, sems2, semw2),
    )

    def process(S, g, t):
        bufr, bufs, bhi, blo, semr, sems, semw = S
        drain_gathers(bufr, bufs, semr, sems)

        @pl.when(t > 0)
        def _():
            drain_write(bhi, blo, semw)

        transpose(bufr, bufs, bhi, blo, VPG)
        write(bhi, blo, g, semw)

    NROUND = (NGRP - 2) // 3
    fire(0, sets[0][0], sets[0][1], sets[0][4], sets[0][5])
    fire(1, sets[1][0], sets[1][1], sets[1][4], sets[1][5])

    def round_body(t, carry):
        g0 = 3 * t
        for k in range(3):
            S = sets[k]
            if k < 2:
                nxt = sets[(k + 2) % 3]
                fire(g0 + k + 2, nxt[0], nxt[1], nxt[4], nxt[5])
            else:
                @pl.when(t < NROUND - 1)
                def _():
                    nxt = sets[1]
                    fire(g0 + 4, nxt[0], nxt[1], nxt[4], nxt[5])
            process(S, g0 + k, t)
        return carry

    lax.fori_loop(0, NROUND, round_body, 0)
    fire(NGRP - 1, sets[1][0], sets[1][1], sets[1][4], sets[1][5])
    process(sets[0], NGRP - 2, 1)
    process(sets[1], NGRP - 1, 1)
    drain_write(bhi0, blo0, semw0)
    drain_write(bhi1, blo1, semw1)
    drain_write(bhi2, blo2, semw2)

    to = NGRP * GROUP_E
    cpr = pltpu.async_copy(prer_hbm.at[colv.at[pl.ds(to, TAIL)]],
                           bufr2.at[pl.ds(0, TAIL)], semr2)
    cps = pltpu.async_copy(pres_hbm.at[rowv.at[pl.ds(to, TAIL)]],
                           bufs2.at[pl.ds(0, TAIL)], sems2)
    cpr.wait()
    cps.wait()
    transpose(bufr2, bufs2, bhi2, blo2, TAIL // LANE)
    dtl = pl.ds(ebase + to, TAIL)
    pltpu.sync_copy(bhi2.at[:, pl.ds(0, TAIL)], ghi_hbm.at[:, dtl])
    pltpu.sync_copy(blo2.at[:, pl.ds(0, TAIL)], glo_hbm.at[:, dtl])


def _gather(prer_p, pres_p, col1, row1):
    mesh = plsc.VectorSubcoreMesh(core_axis_name="c", subcore_axis_name="s")
    fn = pl.kernel(
        _gather_body,
        compiler_params=pltpu.CompilerParams(use_tc_tiling_on_sc=False,
                                             needs_layout_passes=False),
        out_type=(
            jax.ShapeDtypeStruct((8, N_EDGES), jnp.float32),
            jax.ShapeDtypeStruct((8, N_EDGES), jnp.float32),
        ),
        mesh=mesh,
        scratch_types=[
            pltpu.VMEM((EPW,), jnp.int32),
            pltpu.VMEM((EPW,), jnp.int32),
            pltpu.VMEM((GROUP_E, LATENT), jnp.float32),
            pltpu.VMEM((GROUP_E, LATENT), jnp.float32),
            pltpu.VMEM((GROUP_E, LATENT), jnp.float32),
            pltpu.VMEM((GROUP_E, LATENT), jnp.float32),
            pltpu.VMEM((GROUP_E, LATENT), jnp.float32),
            pltpu.VMEM((GROUP_E, LATENT), jnp.float32),
            pltpu.VMEM((8, GROUP_E), jnp.float32),
            pltpu.VMEM((8, GROUP_E), jnp.float32),
            pltpu.VMEM((8, GROUP_E), jnp.float32),
            pltpu.VMEM((8, GROUP_E), jnp.float32),
            pltpu.VMEM((8, GROUP_E), jnp.float32),
            pltpu.VMEM((8, GROUP_E), jnp.float32),
            pltpu.SemaphoreType.DMA,
            pltpu.SemaphoreType.DMA,
            pltpu.SemaphoreType.DMA,
            pltpu.SemaphoreType.DMA,
            pltpu.SemaphoreType.DMA,
            pltpu.SemaphoreType.DMA,
            pltpu.SemaphoreType.DMA,
            pltpu.SemaphoreType.DMA,
            pltpu.SemaphoreType.DMA,
        ],
    )
    prer = prer_p.reshape(N_NODES, LATENT)
    pres = pres_p.reshape(N_NODES, LATENT)
    return fn(prer, pres, col1, row1)


_BLK = 32000


def _mlp_body(et_ref, ghi_ref, glo_ref, w1t_ref, w2t_ref, bb_ref, out_ref):
    t = jnp.dot(w1t_ref[...], et_ref[...], preferred_element_type=jnp.float32)
    g = jnp.concatenate([ghi_ref[...], glo_ref[...]], axis=0)
    h = jnp.maximum(t + g + bb_ref[:, 0:1], 0.0)
    h2 = jnp.dot(w2t_ref[...], h, preferred_element_type=jnp.float32) + bb_ref[:, 1:2]
    h2 = jnp.maximum(h2, 0.0)
    mu = jnp.mean(h2, axis=0, keepdims=True)
    d = h2 - mu
    var = jnp.mean(d * d, axis=0, keepdims=True)
    out_ref[...] = d * lax.rsqrt(var + 1e-5) * bb_ref[:, 2:3] + bb_ref[:, 3:4]


def _mlp(et, ghi, glo, w1t, w2t, bb):
    big = pl.BlockSpec((LATENT, _BLK), lambda i: (0, i))
    half = pl.BlockSpec((8, _BLK), lambda i: (0, i))
    return pl.pallas_call(
        _mlp_body,
        grid=(N_EDGES // _BLK,),
        in_specs=[big, half, half,
                  pl.BlockSpec((LATENT, LATENT), lambda i: (0, 0)),
                  pl.BlockSpec((LATENT, LATENT), lambda i: (0, 0)),
                  pl.BlockSpec((LATENT, 4), lambda i: (0, 0))],
        out_specs=big,
        out_shape=jax.ShapeDtypeStruct((LATENT, N_EDGES), jnp.float32),
    )(et, ghi, glo, w1t, w2t, bb)


def kernel(x, e, u, edge_index, batch, W1, b1, W2, b2, gamma, beta):
    f32 = jnp.float32
    eye8 = jnp.eye(8, dtype=f32)
    w_r = W1[D_EDGE:D_EDGE + D_FEAT]
    w_s = W1[D_EDGE + D_FEAT:D_EDGE + 2 * D_FEAT]
    w_u = W1[D_EDGE + 2 * D_FEAT:]
    wbr = jnp.kron(eye8, w_r)
    wbs = jnp.kron(eye8, w_s)
    m8 = jnp.kron(eye8, jnp.ones((16, 16), f32))
    rmat = jnp.kron(eye8, jnp.ones((1, 16), f32))

    x8 = x.reshape(N_NODES // 8, 8 * D_FEAT)
    b8 = batch.reshape(N_NODES // 8, 8)
    prer_p, pres_p = _node_pre(x8, wbr, wbs, u, w_u, b8, rmat, m8)

    ghi, glo = _gather(prer_p, pres_p, edge_index[1], edge_index[0])

    w1t = W1[:D_EDGE].T
    w2t = W2.T
    bb = jnp.stack([b1, b2, gamma, beta], axis=1)

    outt = _mlp(e.T, ghi, glo, w1t, w2t, bb)
    return outt.T


# speedup vs baseline: 1.5289x; 1.5289x over previous
import functools

import jax
import jax.numpy as jnp
from jax import lax
from jax.experimental import pallas as pl
from jax.experimental.pallas import tpu as pltpu
from jax.experimental.pallas import tpu_sc as plsc

N_NODES = 10000
N_EDGES = 320000
N_GRAPHS = 16
D_FEAT = 128
D_EDGE = 16
LATENT = 16

NC = 2
NS = 16
NW = NC * NS
LANE = 16

EPW = N_EDGES // NW
STEP = 128
G = 3
NGRP = 26
NPAIR = NGRP // 2
GROUP_E = G * STEP
TAIL = EPW - NGRP * GROUP_E
VPG = GROUP_E // LANE


def _node_pre_body(x8_ref, wbr_ref, wbs_ref, u_ref, wu_ref, b8_ref, r_ref,
                   m8_ref, prer_ref, pres_ref):
    x8 = x8_ref[...]
    prer_ref[...] = jnp.dot(x8, wbr_ref[...], preferred_element_type=jnp.float32)
    uw = jnp.dot(u_ref[...], wu_ref[...], preferred_element_type=jnp.float32)
    uw8 = jnp.tile(uw, (8, 8)) * m8_ref[...]
    brep = jnp.dot(b8_ref[...].astype(jnp.float32), r_ref[...],
                   preferred_element_type=jnp.float32)
    g16 = (lax.broadcasted_iota(jnp.int32, (1, 128), 1) % 16).astype(jnp.float32)
    oh = (brep == g16).astype(jnp.float32)
    pres_ref[...] = (jnp.dot(x8, wbs_ref[...], preferred_element_type=jnp.float32)
                     + jnp.dot(oh, uw8, preferred_element_type=jnp.float32))


def _node_pre(x8, wbr, wbs, u, wu, b8, r, m8):
    return pl.pallas_call(
        _node_pre_body,
        out_shape=(
            jax.ShapeDtypeStruct((N_NODES // 8, 128), jnp.float32),
            jax.ShapeDtypeStruct((N_NODES // 8, 128), jnp.float32),
        ),
    )(x8, wbr, wbs, u, wu, b8, r, m8)


def _gather_body(prer_hbm, pres_hbm, col_hbm, row_hbm, ghi_hbm, glo_hbm,
                 colv, rowv, bufr0, bufs0, bufr1, bufs1, bufr2, bufs2,
                 buft0, buft1, buft2,
                 semr0, sems0, semr1, sems1, semr2, sems2,
                 semw0, semw1, semw2):
    wid = lax.axis_index("s") * NC + lax.axis_index("c")
    ebase = wid * EPW
    pltpu.sync_copy(col_hbm.at[pl.ds(ebase, EPW)], colv)
    pltpu.sync_copy(row_hbm.at[pl.ds(ebase, EPW)], rowv)
    lane = lax.broadcasted_iota(jnp.int32, (LANE,), 0)

    def fire(g, bufr, bufs, semr, sems):
        for b in range(G):
            o = g * GROUP_E + b * STEP
            d = pl.ds(b * STEP, STEP)
            pltpu.async_copy(prer_hbm.at[colv.at[pl.ds(o, STEP)]],
                             bufr.at[d], semr)
            pltpu.async_copy(pres_hbm.at[rowv.at[pl.ds(o, STEP)]],
                             bufs.at[d], sems)

    def drain_gathers(bufr, bufs, semr, sems):
        pltpu.make_async_copy(prer_hbm.at[pl.ds(0, GROUP_E)], bufr, semr).wait()
        pltpu.make_async_copy(pres_hbm.at[pl.ds(0, GROUP_E)], bufs, sems).wait()

    def transpose(bufr, bufs, buft, n_edges):
        def edge_body(e):
            v = bufr[e, :] + bufs[e, :]
            ecol = jnp.full((LANE,), e, jnp.int32)
            plsc.store_scatter(buft, [lane, ecol], v)
        plsc.parallel_loop(0, n_edges, unroll=8)(edge_body)

    def write(buft, g, semw):
        d = pl.ds(ebase + g * GROUP_E, GROUP_E)
        ds_e = pl.ds(0, GROUP_E)
        pltpu.async_copy(buft.at[pl.ds(0, 8), ds_e], ghi_hbm.at[:, d], semw)
        pltpu.async_copy(buft.at[pl.ds(8, 8), ds_e], glo_hbm.at[:, d], semw)

    def drain_write(buft, semw):
        d = pl.ds(ebase, GROUP_E)
        ds_e = pl.ds(0, GROUP_E)
        pltpu.make_async_copy(buft.at[pl.ds(0, 8), ds_e],
                              ghi_hbm.at[:, d], semw).wait()
        pltpu.make_async_copy(buft.at[pl.ds(8, 8), ds_e],
                              glo_hbm.at[:, d], semw).wait()

    sets = (
        (bufr0, bufs0, buft0, semr0, sems0, semw0),
        (bufr1, bufs1, buft1, semr1, sems1, semw1),
        (bufr2, bufs2, buft2, semr2, sems2, semw2),
    )

    def process(S, g, t):
        bufr, bufs, buft, semr, sems, semw = S
        drain_gathers(bufr, bufs, semr, sems)

        @pl.when(t > 0)
        def _():
            drain_write(buft, semw)

        transpose(bufr, bufs, buft, GROUP_E)
        write(buft, g, semw)

    NROUND = (NGRP - 2) // 3
    fire(0, sets[0][0], sets[0][1], sets[0][3], sets[0][4])
    fire(1, sets[1][0], sets[1][1], sets[1][3], sets[1][4])

    def round_body(t, carry):
        g0 = 3 * t
        for k in range(3):
            S = sets[k]
            if k < 2:
                nxt = sets[(k + 2) % 3]
                fire(g0 + k + 2, nxt[0], nxt[1], nxt[3], nxt[4])
            else:
                @pl.when(t < NROUND - 1)
                def _():
                    nxt = sets[1]
                    fire(g0 + 4, nxt[0], nxt[1], nxt[3], nxt[4])
            process(S, g0 + k, t)
        return carry

    lax.fori_loop(0, NROUND, round_body, 0)
    fire(NGRP - 1, sets[1][0], sets[1][1], sets[1][3], sets[1][4])
    process(sets[0], NGRP - 2, 1)
    process(sets[1], NGRP - 1, 1)
    drain_write(buft0, semw0)
    drain_write(buft1, semw1)
    drain_write(buft2, semw2)

    to = NGRP * GROUP_E
    cpr = pltpu.async_copy(prer_hbm.at[colv.at[pl.ds(to, TAIL)]],
                           bufr2.at[pl.ds(0, TAIL)], semr2)
    cps = pltpu.async_copy(pres_hbm.at[rowv.at[pl.ds(to, TAIL)]],
                           bufs2.at[pl.ds(0, TAIL)], sems2)
    cpr.wait()
    cps.wait()
    transpose(bufr2, bufs2, buft2, TAIL)
    dtl = pl.ds(ebase + to, TAIL)
    dse = pl.ds(0, TAIL)
    pltpu.sync_copy(buft2.at[pl.ds(0, 8), dse], ghi_hbm.at[:, dtl])
    pltpu.sync_copy(buft2.at[pl.ds(8, 8), dse], glo_hbm.at[:, dtl])


def _gather(prer_p, pres_p, col1, row1):
    mesh = plsc.VectorSubcoreMesh(core_axis_name="c", subcore_axis_name="s")
    fn = pl.kernel(
        _gather_body,
        compiler_params=pltpu.CompilerParams(use_tc_tiling_on_sc=False,
                                             needs_layout_passes=False),
        out_type=(
            jax.ShapeDtypeStruct((8, N_EDGES), jnp.float32),
            jax.ShapeDtypeStruct((8, N_EDGES), jnp.float32),
        ),
        mesh=mesh,
        scratch_types=[
            pltpu.VMEM((EPW,), jnp.int32),
            pltpu.VMEM((EPW,), jnp.int32),
            pltpu.VMEM((GROUP_E, LATENT), jnp.float32),
            pltpu.VMEM((GROUP_E, LATENT), jnp.float32),
            pltpu.VMEM((GROUP_E, LATENT), jnp.float32),
            pltpu.VMEM((GROUP_E, LATENT), jnp.float32),
            pltpu.VMEM((GROUP_E, LATENT), jnp.float32),
            pltpu.VMEM((GROUP_E, LATENT), jnp.float32),
            pltpu.VMEM((LATENT, GROUP_E + 1), jnp.float32),
            pltpu.VMEM((LATENT, GROUP_E + 1), jnp.float32),
            pltpu.VMEM((LATENT, GROUP_E + 1), jnp.float32),
            pltpu.SemaphoreType.DMA,
            pltpu.SemaphoreType.DMA,
            pltpu.SemaphoreType.DMA,
            pltpu.SemaphoreType.DMA,
            pltpu.SemaphoreType.DMA,
            pltpu.SemaphoreType.DMA,
            pltpu.SemaphoreType.DMA,
            pltpu.SemaphoreType.DMA,
            pltpu.SemaphoreType.DMA,
        ],
    )
    prer = prer_p.reshape(N_NODES, LATENT)
    pres = pres_p.reshape(N_NODES, LATENT)
    return fn(prer, pres, col1, row1)


_BLK = 32000


def _mlp_body(et_ref, ghi_ref, glo_ref, w1t_ref, w2t_ref, bb_ref, out_ref):
    t = jnp.dot(w1t_ref[...], et_ref[...], preferred_element_type=jnp.float32)
    g = jnp.concatenate([ghi_ref[...], glo_ref[...]], axis=0)
    h = jnp.maximum(t + g + bb_ref[:, 0:1], 0.0)
    h2 = jnp.dot(w2t_ref[...], h, preferred_element_type=jnp.float32) + bb_ref[:, 1:2]
    h2 = jnp.maximum(h2, 0.0)
    mu = jnp.mean(h2, axis=0, keepdims=True)
    d = h2 - mu
    var = jnp.mean(d * d, axis=0, keepdims=True)
    out_ref[...] = d * lax.rsqrt(var + 1e-5) * bb_ref[:, 2:3] + bb_ref[:, 3:4]


def _mlp(et, ghi, glo, w1t, w2t, bb):
    big = pl.BlockSpec((LATENT, _BLK), lambda i: (0, i))
    half = pl.BlockSpec((8, _BLK), lambda i: (0, i))
    return pl.pallas_call(
        _mlp_body,
        grid=(N_EDGES // _BLK,),
        in_specs=[big, half, half,
                  pl.BlockSpec((LATENT, LATENT), lambda i: (0, 0)),
                  pl.BlockSpec((LATENT, LATENT), lambda i: (0, 0)),
                  pl.BlockSpec((LATENT, 4), lambda i: (0, 0))],
        out_specs=big,
        out_shape=jax.ShapeDtypeStruct((LATENT, N_EDGES), jnp.float32),
    )(et, ghi, glo, w1t, w2t, bb)


def kernel(x, e, u, edge_index, batch, W1, b1, W2, b2, gamma, beta):
    f32 = jnp.float32
    eye8 = jnp.eye(8, dtype=f32)
    w_r = W1[D_EDGE:D_EDGE + D_FEAT]
    w_s = W1[D_EDGE + D_FEAT:D_EDGE + 2 * D_FEAT]
    w_u = W1[D_EDGE + 2 * D_FEAT:]
    wbr = jnp.kron(eye8, w_r)
    wbs = jnp.kron(eye8, w_s)
    m8 = jnp.kron(eye8, jnp.ones((16, 16), f32))
    rmat = jnp.kron(eye8, jnp.ones((1, 16), f32))

    x8 = x.reshape(N_NODES // 8, 8 * D_FEAT)
    b8 = batch.reshape(N_NODES // 8, 8)
    prer_p, pres_p = _node_pre(x8, wbr, wbs, u, w_u, b8, rmat, m8)

    ghi, glo = _gather(prer_p, pres_p, edge_index[1], edge_index[0])

    w1t = W1[:D_EDGE].T
    w2t = W2.T
    bb = jnp.stack([b1, b2, gamma, beta], axis=1)

    outt = _mlp(e.T, ghi, glo, w1t, w2t, bb)
    return outt.T
